# baseline (speedup 1.0000x reference)
import functools

import jax
import jax.numpy as jnp
from jax import lax
from jax.experimental import pallas as pl
from jax.experimental.pallas import tpu as pltpu
from jax.experimental.pallas import tpu_sc as plsc

F = 26
V = 100000
E = 32
B = 4096
GEN = F * E
DOM = 4 * E
GH = 256
TN = 2
NW = 32
BPW = B // NW
RPW = BPW * F
CH = 128
NCH = RPW // CH


def _sc_gather(idx3d, table_flat):
    mesh = plsc.VectorSubcoreMesh(core_axis_name="c", subcore_axis_name="s")
    nc = mesh.num_cores

    @functools.partial(
        pl.kernel,
        out_type=jax.ShapeDtypeStruct((B * F, E), jnp.float32),
        mesh=mesh,
        scratch_types=[
            pltpu.VMEM((NCH, CH), jnp.int32),
            pltpu.VMEM((RPW, E), jnp.float32),
            pltpu.SemaphoreType.DMA,
        ],
        compiler_params=pltpu.CompilerParams(use_tc_tiling_on_sc=False),
    )
    def k(idx_hbm, tbl_hbm, out_hbm, idx_v, rows_v, sem):
        wid = lax.axis_index("s") * nc + lax.axis_index("c")
        pltpu.sync_copy(idx_hbm.at[wid], idx_v)
        copies = []
        for g in range(NCH):
            copies.append(
                pltpu.async_copy(
                    tbl_hbm.at[idx_v.at[g]],
                    rows_v.at[pl.ds(g * CH, CH)],
                    sem,
                )
            )
        for c in copies:
            c.wait()
        pltpu.sync_copy(rows_v, out_hbm.at[pl.ds(wid * RPW, RPW)])

    return k(idx3d, table_flat)


def _dense(emb, gw1, gb1, gw2, gb2, tw1, tb1, tw2, tb2, tw3, tb3):
    BT = 512
    grid = (B // BT,)

    def body(emb_ref, gw1_ref, gb1_ref, gw2_ref, gb2_ref,
             tw1_ref, tb1_ref, tw2_ref, tb2_ref, tw3_ref, tb3_ref, out_ref):
        emb_blk = emb_ref[...]
        h = jnp.dot(emb_blk[:, :DOM], gw1_ref[:DOM, :],
                    preferred_element_type=jnp.float32)
        h = h + jnp.dot(emb_blk, gw1_ref[DOM:, :],
                        preferred_element_type=jnp.float32)
        h = jnp.maximum(h + gb1_ref[...], 0.0)
        g = jnp.dot(h, gw2_ref[...], preferred_element_type=jnp.float32)
        g = 2.0 * jax.nn.sigmoid(g + gb2_ref[...])
        ep = jnp.tile(g, (1, F)) * emb_blk
        outs = []
        for t in range(TN):
            h1 = jnp.dot(ep, tw1_ref[t], preferred_element_type=jnp.float32)
            h1 = jnp.maximum(h1 + tb1_ref[t], 0.0)
            h2 = jnp.dot(h1, tw2_ref[t], preferred_element_type=jnp.float32)
            h2 = jnp.maximum(h2 + tb2_ref[t], 0.0)
            lg = jnp.dot(h2, tw3_ref[t], preferred_element_type=jnp.float32)
            outs.append(lg + tb3_ref[t])
        out_ref[...] = jnp.concatenate(outs, axis=1)

    full = lambda *shape: pl.BlockSpec(shape, lambda i: (0,) * len(shape))
    return pl.pallas_call(
        body,
        grid=grid,
        in_specs=[
            pl.BlockSpec((BT, GEN), lambda i: (i, 0)),
            full(*gw1.shape), full(*gb1.shape), full(*gw2.shape), full(*gb2.shape),
            full(*tw1.shape), full(*tb1.shape), full(*tw2.shape), full(*tb2.shape),
            full(*tw3.shape), full(*tb3.shape),
        ],
        out_specs=pl.BlockSpec((BT, TN), lambda i: (i, 0)),
        out_shape=jax.ShapeDtypeStruct((B, TN), jnp.float32),
        compiler_params=pltpu.CompilerParams(
            dimension_semantics=("arbitrary",),
        ),
    )(emb, gw1, gb1, gw2, gb2, tw1, tb1, tw2, tb2, tw3, tb3)


def kernel(inputs, tables, gate_W1, gate_b1, gate_W2, gate_b2,
           tower_W1, tower_b1, tower_W2, tower_b2, tower_W3, tower_b3):
    flat_idx = inputs.astype(jnp.int32) + (jnp.arange(F, dtype=jnp.int32) * V)[None, :]
    idx3d = flat_idx.reshape(NW, NCH, CH)
    table_flat = tables.reshape(F * V, E)
    emb = _sc_gather(idx3d, table_flat).reshape(B, GEN)
    return _dense(emb, gate_W1, gate_b1, gate_W2, gate_b2,
                  tower_W1, tower_b1, tower_W2, tower_b2, tower_W3, tower_b3)
